# Initial kernel scaffold; baseline (speedup 1.0000x reference)
import jax
import jax.numpy as jnp
from jax.experimental import pallas as pl


def kernel(edge_predictions, edge_index, y_edges, num_nodes):
    raise NotImplementedError("write your pallas kernel here")


# speedup vs baseline: 19.3086x; 19.3086x over previous
import jax
import jax.numpy as jnp
from jax import lax
from jax.experimental import pallas as pl
from jax.experimental.pallas import tpu as pltpu
from jax.experimental.pallas import tpu_sc as plsc

N = 100000
E = 3200000
N_PAD = 100352
NC = 2
NS = 16
NW = NC * NS

BLK = 128
SUP = 8
EB = E // BLK
NSUPER = E // (BLK * SUP)
SLICE = N_PAD // NS

COVERAGE_W = 5.0
TOUR_W = 3.0
DEPOT_W = 2.0
SIM_W = 0.3
FOCAL_ALPHA = 0.25
FOCAL_GAMMA = 2.0


def _sc_body(preds_hbm, dst_hbm, src_hbm, out_hbm,
             acc_in, acc_out, pbuf, prb, dbuf, sbuf, zbuf, sem_ld, sem_sc):
  c = lax.axis_index("c")
  s = lax.axis_index("s")
  wid = c * NS + s

  def _zero(i, _):
    zbuf[pl.ds(i * 16, 16)] = jnp.zeros((16,), jnp.float32)
    return 0
  lax.fori_loop(0, SLICE // 16, _zero, 0, unroll=8)
  pltpu.sync_copy(zbuf, acc_in.at[pl.ds(s * SLICE, SLICE)])
  pltpu.sync_copy(zbuf, acc_out.at[pl.ds(s * SLICE, SLICE)])
  plsc.subcore_barrier()

  sc0 = (wid * NSUPER) // NW
  sc1 = ((wid + 1) * NSUPER) // NW

  def _super(g, _):
    row = g * SUP
    ld0 = pltpu.async_copy(preds_hbm.at[pl.ds(row, SUP), :], pbuf, sem_ld)
    ld1 = pltpu.async_copy(dst_hbm.at[pl.ds(row, SUP), :], dbuf, sem_ld)
    ld2 = pltpu.async_copy(src_hbm.at[pl.ds(row, SUP), :], sbuf, sem_ld)
    ld0.wait()
    ld1.wait()
    ld2.wait()

    def _sig(k, _):
      r = k // (BLK // 16)
      q = (k % (BLK // 16)) * 16
      x = pbuf[r, pl.ds(q, 16)]
      prb[r, pl.ds(q, 16)] = 1.0 / (1.0 + jnp.exp(-x))
      return 0
    lax.fori_loop(0, SUP * (BLK // 16), _sig, 0, unroll=8)

    descs = []
    for r in range(SUP):
      descs.append(pltpu.async_copy(
          prb.at[r], acc_in.at[dbuf.at[r]], sem_sc, add=True))
      descs.append(pltpu.async_copy(
          prb.at[r], acc_out.at[sbuf.at[r]], sem_sc, add=True))
    for d in descs:
      d.wait()
    return 0

  lax.fori_loop(sc0, sc1, _super, 0)
  plsc.subcore_barrier()

  sl = pl.ds(s * SLICE, SLICE)
  pltpu.sync_copy(acc_in.at[sl], out_hbm.at[c, 0, sl])
  pltpu.sync_copy(acc_out.at[sl], out_hbm.at[c, 1, sl])


def _sc_segment_sums(preds2d, dst2d, src2d):
  mesh = plsc.VectorSubcoreMesh(core_axis_name="c", subcore_axis_name="s")
  f = pl.kernel(
      _sc_body,
      out_type=jax.ShapeDtypeStruct((NC, 2, N_PAD), jnp.float32),
      mesh=mesh,
      scratch_types=[
          pltpu.VMEM_SHARED((N_PAD,), jnp.float32),
          pltpu.VMEM_SHARED((N_PAD,), jnp.float32),
          pltpu.VMEM((SUP, BLK), jnp.float32),
          pltpu.VMEM((SUP, BLK), jnp.float32),
          pltpu.VMEM((SUP, BLK), jnp.int32),
          pltpu.VMEM((SUP, BLK), jnp.int32),
          pltpu.VMEM((SLICE,), jnp.float32),
          pltpu.SemaphoreType.DMA,
          pltpu.SemaphoreType.DMA,
      ],
  )
  return f(preds2d, dst2d, src2d)


TC_ROWS = 1000
TC_STEPS = EB // TC_ROWS
ACC_ROWS = N_PAD // 128


def _tc_body(preds_ref, y_ref, planes_ref, out_ref, acc):
  i = pl.program_id(0)

  @pl.when(i == 0)
  def _init():
    in_sum = planes_ref[0, 0] + planes_ref[1, 0]
    out_sum = planes_ref[0, 1] + planes_ref[1, 1]
    n = (lax.broadcasted_iota(jnp.int32, (ACC_ROWS, 128), 0) * 128
         + lax.broadcasted_iota(jnp.int32, (ACC_ROWS, 128), 1))
    customer = jnp.logical_and(n >= 1, n < N)
    zero = jnp.zeros_like(in_sum)
    cov = jnp.sum(jnp.where(customer, (in_sum - 1.0) ** 2, zero)
                  + jnp.where(customer, (out_sum - 1.0) ** 2, zero))
    diff = in_sum - out_sum
    tour = jnp.sum(diff * diff)
    depot = diff[0, 0] * diff[0, 0]
    acc[0] = (COVERAGE_W * cov / (2.0 * (N - 1))
              + TOUR_W * tour / N
              + DEPOT_W * depot)
    acc[1] = 0.0

  x = preds_ref[...]
  t = y_ref[...]
  bce = jnp.maximum(x, 0.0) - x * t + jnp.log1p(jnp.exp(-jnp.abs(x)))
  probs = jax.nn.sigmoid(x)
  p_t = probs * t + (1.0 - probs) * (1.0 - t)
  alpha_t = FOCAL_ALPHA * t + (1.0 - FOCAL_ALPHA) * (1.0 - t)
  w = 1.0 - p_t
  acc[1] += jnp.sum(alpha_t * (w * w) * bce)

  @pl.when(i == TC_STEPS - 1)
  def _fin():
    out_ref[0, 0] = acc[0] + SIM_W * acc[1] / E


def _tc_combine(preds2d, y2d, planes):
  return pl.pallas_call(
      _tc_body,
      grid=(TC_STEPS,),
      in_specs=[
          pl.BlockSpec((TC_ROWS, 128), lambda i: (i, 0)),
          pl.BlockSpec((TC_ROWS, 128), lambda i: (i, 0)),
          pl.BlockSpec((NC, 2, ACC_ROWS, 128), lambda i: (0, 0, 0, 0)),
      ],
      out_specs=pl.BlockSpec(memory_space=pltpu.SMEM),
      out_shape=jax.ShapeDtypeStruct((1, 1), jnp.float32),
      scratch_shapes=[pltpu.SMEM((2,), jnp.float32)],
  )(preds2d, y2d, planes)


def kernel(edge_predictions, edge_index, y_edges, num_nodes):
  preds2d = edge_predictions.reshape(EB, BLK)
  dst2d = edge_index[1].reshape(EB, BLK)
  src2d = edge_index[0].reshape(EB, BLK)
  y2d = y_edges.reshape(EB, BLK)

  planes = _sc_segment_sums(preds2d, dst2d, src2d)
  planes4 = planes.reshape(NC, 2, ACC_ROWS, 128)
  total = _tc_combine(preds2d, y2d, planes4)
  return total.reshape(())


# speedup vs baseline: 20.9983x; 1.0875x over previous
import jax
import jax.numpy as jnp
from jax import lax
from jax.experimental import pallas as pl
from jax.experimental.pallas import tpu as pltpu
from jax.experimental.pallas import tpu_sc as plsc

N = 100000
E = 3200000
N_PAD = 100352
NC = 2
NS = 16
NW = NC * NS

BLK = 128
CH = 3200
EB = E // BLK
NCH = E // CH

COVERAGE_W = 5.0
TOUR_W = 3.0
DEPOT_W = 2.0
SIM_W = 0.3
FOCAL_ALPHA = 0.25
FOCAL_GAMMA = 2.0


def _sc_body(preds_hbm, idx_hbm, out_hbm,
             acc, pbuf0, pbuf1, ibuf0, ibuf1, sem0, sem1):
  c = lax.axis_index("c")
  s = lax.axis_index("s")
  wid = c * NS + s
  sel = 1 - c

  def _zero(i, _):
    acc[pl.ds(i * 16, 16)] = jnp.zeros((16,), jnp.float32)
    return 0
  lax.fori_loop(0, N_PAD // 16, _zero, 0, unroll=16)

  g0 = (s * NCH) // NS
  g1 = ((s + 1) * NCH) // NS

  def _load(g, pbuf, ibuf, sem):
    pltpu.async_copy(preds_hbm.at[pl.ds(g * CH, CH)], pbuf, sem)
    pltpu.async_copy(idx_hbm.at[pl.ds(sel * E + g * CH, CH)], ibuf, sem)

  def _wait(pbuf, ibuf, sem):
    pltpu.make_async_copy(preds_hbm.at[pl.ds(0, CH)], pbuf, sem).wait()
    pltpu.make_async_copy(idx_hbm.at[pl.ds(0, CH)], ibuf, sem).wait()

  def _compute(pbuf, ibuf):
    def _step(k, _):
      x = pbuf[pl.ds(k * 16, 16)]
      p = 1.0 / (1.0 + jnp.exp(-x))
      ids = ibuf[pl.ds(k * 16, 16)]
      plsc.addupdate_scatter(acc, [ids], p)
      return 0
    lax.fori_loop(0, CH // 16, _step, 0, unroll=8)

  _load(g0, pbuf0, ibuf0, sem0)

  def _pair(k, _):
    a = g0 + 2 * k
    b = a + 1

    @pl.when(b < g1)
    def _():
      _load(b, pbuf1, ibuf1, sem1)

    _wait(pbuf0, ibuf0, sem0)
    _compute(pbuf0, ibuf0)

    @pl.when(a + 2 < g1)
    def _():
      _load(a + 2, pbuf0, ibuf0, sem0)

    @pl.when(b < g1)
    def _():
      _wait(pbuf1, ibuf1, sem1)
      _compute(pbuf1, ibuf1)
    return 0

  lax.fori_loop(0, (g1 - g0 + 1) // 2, _pair, 0)

  pltpu.sync_copy(acc, out_hbm.at[wid, 0])


def _sc_segment_sums(preds1d, eidx1d):
  mesh = plsc.VectorSubcoreMesh(core_axis_name="c", subcore_axis_name="s")
  f = pl.kernel(
      _sc_body,
      out_type=jax.ShapeDtypeStruct((NW, 1, N_PAD), jnp.float32),
      mesh=mesh,
      compiler_params=pltpu.CompilerParams(needs_layout_passes=False),
      scratch_types=[
          pltpu.VMEM((N_PAD,), jnp.float32),
          pltpu.VMEM((CH,), jnp.float32),
          pltpu.VMEM((CH,), jnp.float32),
          pltpu.VMEM((CH,), jnp.int32),
          pltpu.VMEM((CH,), jnp.int32),
          pltpu.SemaphoreType.DMA,
          pltpu.SemaphoreType.DMA,
      ],
  )
  return f(preds1d, eidx1d)


TC_ROWS = 1000
TC_STEPS = EB // TC_ROWS


def _focal_body(preds_ref, y_ref, out_ref, acc):
  i = pl.program_id(0)

  @pl.when(i == 0)
  def _():
    acc[0] = 0.0

  x = preds_ref[...]
  t = y_ref[...]
  bce = jnp.maximum(x, 0.0) - x * t + jnp.log1p(jnp.exp(-jnp.abs(x)))
  probs = jax.nn.sigmoid(x)
  p_t = probs * t + (1.0 - probs) * (1.0 - t)
  alpha_t = FOCAL_ALPHA * t + (1.0 - FOCAL_ALPHA) * (1.0 - t)
  w = 1.0 - p_t
  acc[0] += jnp.sum(alpha_t * (w * w) * bce)

  @pl.when(i == TC_STEPS - 1)
  def _():
    out_ref[0, 0] = acc[0]


def _tc_focal(preds2d, y2d):
  return pl.pallas_call(
      _focal_body,
      grid=(TC_STEPS,),
      in_specs=[
          pl.BlockSpec((TC_ROWS, 128), lambda i: (i, 0)),
          pl.BlockSpec((TC_ROWS, 128), lambda i: (i, 0)),
      ],
      out_specs=pl.BlockSpec(memory_space=pltpu.SMEM),
      out_shape=jax.ShapeDtypeStruct((1, 1), jnp.float32),
      scratch_shapes=[pltpu.SMEM((1,), jnp.float32)],
  )(preds2d, y2d)


CB_ROWS = 112
CB_STEPS = N_PAD // 128 // CB_ROWS


def _combine_body(planes_ref, focal_ref, out_ref, acc):
  j = pl.program_id(0)

  @pl.when(j == 0)
  def _():
    acc[0] = 0.0
    acc[1] = 0.0
    acc[2] = 0.0

  in_s = planes_ref[0]
  out_s = planes_ref[NS]
  for k in range(1, NS):
    in_s = in_s + planes_ref[k]
    out_s = out_s + planes_ref[NS + k]

  n = (lax.broadcasted_iota(jnp.int32, (CB_ROWS, 128), 0) * 128
       + lax.broadcasted_iota(jnp.int32, (CB_ROWS, 128), 1)
       + j * (CB_ROWS * 128))
  customer = jnp.logical_and(n >= 1, n < N)
  zero = jnp.zeros_like(in_s)
  acc[0] += jnp.sum(jnp.where(customer, (in_s - 1.0) ** 2, zero)
                    + jnp.where(customer, (out_s - 1.0) ** 2, zero))
  diff = in_s - out_s
  acc[1] += jnp.sum(diff * diff)

  @pl.when(j == 0)
  def _():
    acc[2] = diff[0, 0] * diff[0, 0]

  @pl.when(j == CB_STEPS - 1)
  def _():
    out_ref[0, 0] = (COVERAGE_W * acc[0] / (2.0 * (N - 1))
                     + TOUR_W * acc[1] / N
                     + DEPOT_W * acc[2]
                     + SIM_W * focal_ref[0, 0] / E)


def _tc_combine(planes3, focal):
  return pl.pallas_call(
      _combine_body,
      grid=(CB_STEPS,),
      in_specs=[
          pl.BlockSpec((NW, CB_ROWS, 128), lambda j: (0, j, 0)),
          pl.BlockSpec(memory_space=pltpu.SMEM),
      ],
      out_specs=pl.BlockSpec(memory_space=pltpu.SMEM),
      out_shape=jax.ShapeDtypeStruct((1, 1), jnp.float32),
      scratch_shapes=[pltpu.SMEM((4,), jnp.float32)],
  )(planes3, focal)


def kernel(edge_predictions, edge_index, y_edges, num_nodes):
  preds2d = edge_predictions.reshape(EB, BLK)
  eidx1d = edge_index.reshape(2 * E)
  y2d = y_edges.reshape(EB, BLK)

  planes = _sc_segment_sums(edge_predictions, eidx1d)
  focal = _tc_focal(preds2d, y2d)
  planes3 = planes.reshape(NW, N_PAD // 128, 128)
  total = _tc_combine(planes3, focal)
  return total.reshape(())


# speedup vs baseline: 41.8489x; 1.9930x over previous
import jax
import jax.numpy as jnp
from jax import lax
from jax.experimental import pallas as pl
from jax.experimental.pallas import tpu as pltpu
from jax.experimental.pallas import tpu_sc as plsc

N = 100000
E = 3200000
N_PAD = 100352
NC = 2
NS = 16
NW = NC * NS

BLK = 128
SUP = 40
EB = E // BLK
NCH = EB // SUP

COVERAGE_W = 5.0
TOUR_W = 3.0
DEPOT_W = 2.0
SIM_W = 0.3
FOCAL_ALPHA = 0.25
FOCAL_GAMMA = 2.0


TC_ROWS = 1000
TC_STEPS = EB // TC_ROWS


def _focal_body(preds_ref, y_ref, out_ref, probs_ref, acc):
  i = pl.program_id(0)

  @pl.when(i == 0)
  def _():
    acc[0] = 0.0

  x = preds_ref[...]
  t = y_ref[...]
  bce = jnp.maximum(x, 0.0) - x * t + jnp.log1p(jnp.exp(-jnp.abs(x)))
  probs = jax.nn.sigmoid(x)
  probs_ref[...] = probs
  p_t = probs * t + (1.0 - probs) * (1.0 - t)
  alpha_t = FOCAL_ALPHA * t + (1.0 - FOCAL_ALPHA) * (1.0 - t)
  w = 1.0 - p_t
  acc[0] += jnp.sum(alpha_t * (w * w) * bce)

  @pl.when(i == TC_STEPS - 1)
  def _():
    out_ref[0, 0] = acc[0]


def _tc_focal(preds2d, y2d):
  return pl.pallas_call(
      _focal_body,
      grid=(TC_STEPS,),
      in_specs=[
          pl.BlockSpec((TC_ROWS, 128), lambda i: (i, 0)),
          pl.BlockSpec((TC_ROWS, 128), lambda i: (i, 0)),
      ],
      out_specs=[
          pl.BlockSpec(memory_space=pltpu.SMEM),
          pl.BlockSpec((TC_ROWS, 128), lambda i: (i, 0)),
      ],
      out_shape=[
          jax.ShapeDtypeStruct((1, 1), jnp.float32),
          jax.ShapeDtypeStruct((EB, BLK), jnp.float32),
      ],
      scratch_shapes=[pltpu.SMEM((1,), jnp.float32)],
  )(preds2d, y2d)


def _sc_body(probs_hbm, idx_hbm, out_hbm,
             acc, pbuf0, pbuf1, ibuf0, ibuf1, sem0, sem1):
  c = lax.axis_index("c")
  s = lax.axis_index("s")
  wid = c * NS + s
  sel = 1 - c

  def _zero(i, _):
    acc[pl.ds(i * 16, 16)] = jnp.zeros((16,), jnp.float32)
    return 0
  lax.fori_loop(0, N_PAD // 16, _zero, 0, unroll=16)

  g0 = (s * NCH) // NS
  g1 = ((s + 1) * NCH) // NS

  def _load(g, pbuf, ibuf, sem):
    row = g * SUP
    pltpu.async_copy(probs_hbm.at[pl.ds(row, SUP), :], pbuf, sem)
    pltpu.async_copy(idx_hbm.at[sel, pl.ds(row, SUP), :], ibuf, sem)

  def _wait(pbuf, ibuf, sem):
    pltpu.make_async_copy(probs_hbm.at[pl.ds(0, SUP), :], pbuf, sem).wait()
    pltpu.make_async_copy(idx_hbm.at[0, pl.ds(0, SUP), :], ibuf, sem).wait()

  def _compute(pbuf, ibuf):
    def _step(k, _):
      r = k // (BLK // 16)
      q = (k % (BLK // 16)) * 16
      p = pbuf[r, pl.ds(q, 16)]
      ids = ibuf[r, pl.ds(q, 16)]
      plsc.addupdate_scatter(acc, [ids], p)
      return 0
    lax.fori_loop(0, SUP * (BLK // 16), _step, 0, unroll=8)

  _load(g0, pbuf0, ibuf0, sem0)

  def _pair(k, _):
    a = g0 + 2 * k
    b = a + 1

    @pl.when(b < g1)
    def _():
      _load(b, pbuf1, ibuf1, sem1)

    _wait(pbuf0, ibuf0, sem0)
    _compute(pbuf0, ibuf0)

    @pl.when(a + 2 < g1)
    def _():
      _load(a + 2, pbuf0, ibuf0, sem0)

    @pl.when(b < g1)
    def _():
      _wait(pbuf1, ibuf1, sem1)
      _compute(pbuf1, ibuf1)
    return 0

  lax.fori_loop(0, (g1 - g0 + 1) // 2, _pair, 0)

  pltpu.sync_copy(acc, out_hbm.at[wid, 0])


def _sc_segment_sums(probs2d, idx3d):
  mesh = plsc.VectorSubcoreMesh(core_axis_name="c", subcore_axis_name="s")
  f = pl.kernel(
      _sc_body,
      out_type=jax.ShapeDtypeStruct((NW, 1, N_PAD), jnp.float32),
      mesh=mesh,
      compiler_params=pltpu.CompilerParams(needs_layout_passes=False),
      scratch_types=[
          pltpu.VMEM((N_PAD,), jnp.float32),
          pltpu.VMEM((SUP, BLK), jnp.float32),
          pltpu.VMEM((SUP, BLK), jnp.float32),
          pltpu.VMEM((SUP, BLK), jnp.int32),
          pltpu.VMEM((SUP, BLK), jnp.int32),
          pltpu.SemaphoreType.DMA,
          pltpu.SemaphoreType.DMA,
      ],
  )
  return f(probs2d, idx3d)


CB_ROWS = 112
CB_STEPS = N_PAD // 128 // CB_ROWS


def _combine_body(planes_ref, focal_ref, out_ref, acc):
  j = pl.program_id(0)

  @pl.when(j == 0)
  def _():
    acc[0] = 0.0
    acc[1] = 0.0
    acc[2] = 0.0

  in_s = planes_ref[0]
  out_s = planes_ref[NS]
  for k in range(1, NS):
    in_s = in_s + planes_ref[k]
    out_s = out_s + planes_ref[NS + k]

  n = (lax.broadcasted_iota(jnp.int32, (CB_ROWS, 128), 0) * 128
       + lax.broadcasted_iota(jnp.int32, (CB_ROWS, 128), 1)
       + j * (CB_ROWS * 128))
  customer = jnp.logical_and(n >= 1, n < N)
  zero = jnp.zeros_like(in_s)
  acc[0] += jnp.sum(jnp.where(customer, (in_s - 1.0) ** 2, zero)
                    + jnp.where(customer, (out_s - 1.0) ** 2, zero))
  diff = in_s - out_s
  acc[1] += jnp.sum(diff * diff)

  @pl.when(j == 0)
  def _():
    acc[2] = diff[0, 0] * diff[0, 0]

  @pl.when(j == CB_STEPS - 1)
  def _():
    out_ref[0, 0] = (COVERAGE_W * acc[0] / (2.0 * (N - 1))
                     + TOUR_W * acc[1] / N
                     + DEPOT_W * acc[2]
                     + SIM_W * focal_ref[0, 0] / E)


def _tc_combine(planes3, focal):
  return pl.pallas_call(
      _combine_body,
      grid=(CB_STEPS,),
      in_specs=[
          pl.BlockSpec((NW, CB_ROWS, 128), lambda j: (0, j, 0)),
          pl.BlockSpec(memory_space=pltpu.SMEM),
      ],
      out_specs=pl.BlockSpec(memory_space=pltpu.SMEM),
      out_shape=jax.ShapeDtypeStruct((1, 1), jnp.float32),
      scratch_shapes=[pltpu.SMEM((4,), jnp.float32)],
  )(planes3, focal)


def kernel(edge_predictions, edge_index, y_edges, num_nodes):
  preds2d = edge_predictions.reshape(EB, BLK)
  idx3d = edge_index.reshape(2, EB, BLK)
  y2d = y_edges.reshape(EB, BLK)

  focal, probs2d = _tc_focal(preds2d, y2d)
  planes = _sc_segment_sums(probs2d, idx3d)
  planes3 = planes.reshape(NW, N_PAD // 128, 128)
  total = _tc_combine(planes3, focal)
  return total.reshape(())


# speedup vs baseline: 41.9495x; 1.0024x over previous
import jax
import jax.numpy as jnp
from jax import lax
from jax.experimental import pallas as pl
from jax.experimental.pallas import tpu as pltpu
from jax.experimental.pallas import tpu_sc as plsc

N = 100000
E = 3200000
N_PAD = 100352
NC = 2
NS = 16
NW = NC * NS

BLK = 128
SUP = 40
EB = E // BLK
NCH = EB // SUP

COVERAGE_W = 5.0
TOUR_W = 3.0
DEPOT_W = 2.0
SIM_W = 0.3
FOCAL_ALPHA = 0.25
FOCAL_GAMMA = 2.0


TC_ROWS = 1000
TC_STEPS = EB // TC_ROWS


def _focal_body(preds_ref, y_ref, out_ref, probs_ref, acc):
  i = pl.program_id(0)

  @pl.when(i == 0)
  def _():
    acc[0] = 0.0

  x = preds_ref[...]
  t = y_ref[...]
  bce = jnp.maximum(x, 0.0) - x * t + jnp.log1p(jnp.exp(-jnp.abs(x)))
  probs = jax.nn.sigmoid(x)
  probs_ref[...] = probs
  p_t = probs * t + (1.0 - probs) * (1.0 - t)
  alpha_t = FOCAL_ALPHA * t + (1.0 - FOCAL_ALPHA) * (1.0 - t)
  w = 1.0 - p_t
  acc[0] += jnp.sum(alpha_t * (w * w) * bce)

  @pl.when(i == TC_STEPS - 1)
  def _():
    out_ref[0, 0] = acc[0]


def _tc_focal(preds2d, y2d):
  return pl.pallas_call(
      _focal_body,
      grid=(TC_STEPS,),
      in_specs=[
          pl.BlockSpec((TC_ROWS, 128), lambda i: (i, 0)),
          pl.BlockSpec((TC_ROWS, 128), lambda i: (i, 0)),
      ],
      out_specs=[
          pl.BlockSpec(memory_space=pltpu.SMEM),
          pl.BlockSpec((TC_ROWS, 128), lambda i: (i, 0)),
      ],
      out_shape=[
          jax.ShapeDtypeStruct((1, 1), jnp.float32),
          jax.ShapeDtypeStruct((EB, BLK), jnp.float32),
      ],
      scratch_shapes=[pltpu.SMEM((1,), jnp.float32)],
  )(preds2d, y2d)


def _sc_body(probs_hbm, idx_hbm, out_hbm,
             acc, pbuf0, pbuf1, ibuf0, ibuf1, sem0, sem1):
  c = lax.axis_index("c")
  s = lax.axis_index("s")
  wid = c * NS + s
  sel = 1 - c

  def _zero(i, _):
    acc[pl.ds(i * 16, 16)] = jnp.zeros((16,), jnp.float32)
    return 0
  lax.fori_loop(0, N_PAD // 16, _zero, 0, unroll=16)

  g0 = (s * NCH) // NS
  g1 = ((s + 1) * NCH) // NS

  def _load(g, pbuf, ibuf, sem):
    row = g * SUP
    pltpu.async_copy(probs_hbm.at[pl.ds(row, SUP), :], pbuf, sem)
    pltpu.async_copy(idx_hbm.at[sel, pl.ds(row, SUP), :], ibuf, sem)

  def _wait(pbuf, ibuf, sem):
    pltpu.make_async_copy(probs_hbm.at[pl.ds(0, SUP), :], pbuf, sem).wait()
    pltpu.make_async_copy(idx_hbm.at[0, pl.ds(0, SUP), :], ibuf, sem).wait()

  def _compute(pbuf, ibuf):
    def _row(r, _):
      for j in range(BLK // 16):
        p = pbuf[r, pl.ds(j * 16, 16)]
        ids = ibuf[r, pl.ds(j * 16, 16)]
        plsc.addupdate_scatter(acc, [ids], p)
      return 0
    lax.fori_loop(0, SUP, _row, 0, unroll=2)

  _load(g0, pbuf0, ibuf0, sem0)

  def _pair(k, _):
    a = g0 + 2 * k
    b = a + 1

    @pl.when(b < g1)
    def _():
      _load(b, pbuf1, ibuf1, sem1)

    _wait(pbuf0, ibuf0, sem0)
    _compute(pbuf0, ibuf0)

    @pl.when(a + 2 < g1)
    def _():
      _load(a + 2, pbuf0, ibuf0, sem0)

    @pl.when(b < g1)
    def _():
      _wait(pbuf1, ibuf1, sem1)
      _compute(pbuf1, ibuf1)
    return 0

  lax.fori_loop(0, (g1 - g0 + 1) // 2, _pair, 0)

  pltpu.sync_copy(acc, out_hbm.at[wid, 0])


def _sc_segment_sums(probs2d, idx3d):
  mesh = plsc.VectorSubcoreMesh(core_axis_name="c", subcore_axis_name="s")
  f = pl.kernel(
      _sc_body,
      out_type=jax.ShapeDtypeStruct((NW, 1, N_PAD), jnp.float32),
      mesh=mesh,
      compiler_params=pltpu.CompilerParams(needs_layout_passes=False),
      scratch_types=[
          pltpu.VMEM((N_PAD,), jnp.float32),
          pltpu.VMEM((SUP, BLK), jnp.float32),
          pltpu.VMEM((SUP, BLK), jnp.float32),
          pltpu.VMEM((SUP, BLK), jnp.int32),
          pltpu.VMEM((SUP, BLK), jnp.int32),
          pltpu.SemaphoreType.DMA,
          pltpu.SemaphoreType.DMA,
      ],
  )
  return f(probs2d, idx3d)


CB_ROWS = 112
CB_STEPS = N_PAD // 128 // CB_ROWS


def _combine_body(planes_ref, focal_ref, out_ref, acc):
  j = pl.program_id(0)

  @pl.when(j == 0)
  def _():
    acc[0] = 0.0
    acc[1] = 0.0
    acc[2] = 0.0

  in_s = planes_ref[0]
  out_s = planes_ref[NS]
  for k in range(1, NS):
    in_s = in_s + planes_ref[k]
    out_s = out_s + planes_ref[NS + k]

  n = (lax.broadcasted_iota(jnp.int32, (CB_ROWS, 128), 0) * 128
       + lax.broadcasted_iota(jnp.int32, (CB_ROWS, 128), 1)
       + j * (CB_ROWS * 128))
  customer = jnp.logical_and(n >= 1, n < N)
  zero = jnp.zeros_like(in_s)
  acc[0] += jnp.sum(jnp.where(customer, (in_s - 1.0) ** 2, zero)
                    + jnp.where(customer, (out_s - 1.0) ** 2, zero))
  diff = in_s - out_s
  acc[1] += jnp.sum(diff * diff)

  @pl.when(j == 0)
  def _():
    acc[2] = diff[0, 0] * diff[0, 0]

  @pl.when(j == CB_STEPS - 1)
  def _():
    out_ref[0, 0] = (COVERAGE_W * acc[0] / (2.0 * (N - 1))
                     + TOUR_W * acc[1] / N
                     + DEPOT_W * acc[2]
                     + SIM_W * focal_ref[0, 0] / E)


def _tc_combine(planes3, focal):
  return pl.pallas_call(
      _combine_body,
      grid=(CB_STEPS,),
      in_specs=[
          pl.BlockSpec((NW, CB_ROWS, 128), lambda j: (0, j, 0)),
          pl.BlockSpec(memory_space=pltpu.SMEM),
      ],
      out_specs=pl.BlockSpec(memory_space=pltpu.SMEM),
      out_shape=jax.ShapeDtypeStruct((1, 1), jnp.float32),
      scratch_shapes=[pltpu.SMEM((4,), jnp.float32)],
  )(planes3, focal)


def kernel(edge_predictions, edge_index, y_edges, num_nodes):
  preds2d = edge_predictions.reshape(EB, BLK)
  idx3d = edge_index.reshape(2, EB, BLK)
  y2d = y_edges.reshape(EB, BLK)

  focal, probs2d = _tc_focal(preds2d, y2d)
  planes = _sc_segment_sums(probs2d, idx3d)
  planes3 = planes.reshape(NW, N_PAD // 128, 128)
  total = _tc_combine(planes3, focal)
  return total.reshape(())


# speedup vs baseline: 41.9936x; 1.0011x over previous
import jax
import jax.numpy as jnp
from jax import lax
from jax.experimental import pallas as pl
from jax.experimental.pallas import tpu as pltpu
from jax.experimental.pallas import tpu_sc as plsc

N = 100000
E = 3200000
N_PAD = 100352
NC = 2
NS = 16
NW = NC * NS

BLK = 128
SUP = 40
EB = E // BLK
NCH = EB // SUP

COVERAGE_W = 5.0
TOUR_W = 3.0
DEPOT_W = 2.0
SIM_W = 0.3
FOCAL_ALPHA = 0.25
FOCAL_GAMMA = 2.0


TC_ROWS = 1000
TC_STEPS = EB // TC_ROWS


def _focal_body(preds_ref, y_ref, out_ref, probs_ref, acc):
  i = pl.program_id(0)

  @pl.when(i == 0)
  def _():
    acc[0] = 0.0

  x = preds_ref[...]
  t = y_ref[...]
  bce = jnp.maximum(x, 0.0) - x * t + jnp.log1p(jnp.exp(-jnp.abs(x)))
  probs = jax.nn.sigmoid(x)
  probs_ref[...] = probs
  p_t = probs * t + (1.0 - probs) * (1.0 - t)
  alpha_t = FOCAL_ALPHA * t + (1.0 - FOCAL_ALPHA) * (1.0 - t)
  w = 1.0 - p_t
  acc[0] += jnp.sum(alpha_t * (w * w) * bce)

  @pl.when(i == TC_STEPS - 1)
  def _():
    out_ref[0, 0] = acc[0]


def _tc_focal(preds2d, y2d):
  return pl.pallas_call(
      _focal_body,
      grid=(TC_STEPS,),
      in_specs=[
          pl.BlockSpec((TC_ROWS, 128), lambda i: (i, 0)),
          pl.BlockSpec((TC_ROWS, 128), lambda i: (i, 0)),
      ],
      out_specs=[
          pl.BlockSpec(memory_space=pltpu.SMEM),
          pl.BlockSpec((TC_ROWS, 128), lambda i: (i, 0)),
      ],
      out_shape=[
          jax.ShapeDtypeStruct((1, 1), jnp.float32),
          jax.ShapeDtypeStruct((EB, BLK), jnp.float32),
      ],
      scratch_shapes=[pltpu.SMEM((1,), jnp.float32)],
  )(preds2d, y2d)


def _sc_body(probs_hbm, idx_hbm, out_hbm,
             acc, pbuf0, pbuf1, ibuf0, ibuf1, sem0, sem1):
  c = lax.axis_index("c")
  s = lax.axis_index("s")
  wid = c * NS + s
  sel = 1 - c

  def _zero(i, _):
    acc[pl.ds(i * 16, 16)] = jnp.zeros((16,), jnp.float32)
    return 0
  lax.fori_loop(0, N_PAD // 16, _zero, 0, unroll=16)

  g0 = (s * NCH) // NS
  g1 = ((s + 1) * NCH) // NS

  def _load(g, pbuf, ibuf, sem):
    row = g * SUP
    pltpu.async_copy(probs_hbm.at[pl.ds(row, SUP), :], pbuf, sem)
    pltpu.async_copy(idx_hbm.at[sel, pl.ds(row, SUP), :], ibuf, sem)

  def _wait(pbuf, ibuf, sem):
    pltpu.make_async_copy(probs_hbm.at[pl.ds(0, SUP), :], pbuf, sem).wait()
    pltpu.make_async_copy(idx_hbm.at[0, pl.ds(0, SUP), :], ibuf, sem).wait()

  def _compute(pbuf, ibuf):
    def _row(r, _):
      for j in range(BLK // 16):
        p = pbuf[r, pl.ds(j * 16, 16)]
        ids = ibuf[r, pl.ds(j * 16, 16)]
        plsc.addupdate_scatter(acc, [ids], p)
      return 0
    lax.fori_loop(0, SUP, _row, 0, unroll=2)

  _load(g0, pbuf0, ibuf0, sem0)

  def _pair(k, _):
    a = g0 + 2 * k
    b = a + 1

    @pl.when(b < g1)
    def _():
      _load(b, pbuf1, ibuf1, sem1)

    _wait(pbuf0, ibuf0, sem0)
    _compute(pbuf0, ibuf0)

    @pl.when(a + 2 < g1)
    def _():
      _load(a + 2, pbuf0, ibuf0, sem0)

    @pl.when(b < g1)
    def _():
      _wait(pbuf1, ibuf1, sem1)
      _compute(pbuf1, ibuf1)
    return 0

  lax.fori_loop(0, (g1 - g0 + 1) // 2, _pair, 0)

  pltpu.sync_copy(acc, out_hbm.at[wid, 0])


def _sc_segment_sums(probs2d, idx3d):
  mesh = plsc.VectorSubcoreMesh(core_axis_name="c", subcore_axis_name="s")
  f = pl.kernel(
      _sc_body,
      out_type=jax.ShapeDtypeStruct((NW, 1, N_PAD), jnp.float32),
      mesh=mesh,
      compiler_params=pltpu.CompilerParams(
          needs_layout_passes=False, use_tc_tiling_on_sc=True),
      scratch_types=[
          pltpu.VMEM((N_PAD,), jnp.float32),
          pltpu.VMEM((SUP, BLK), jnp.float32),
          pltpu.VMEM((SUP, BLK), jnp.float32),
          pltpu.VMEM((SUP, BLK), jnp.int32),
          pltpu.VMEM((SUP, BLK), jnp.int32),
          pltpu.SemaphoreType.DMA,
          pltpu.SemaphoreType.DMA,
      ],
  )
  return f(probs2d, idx3d)


CB_ROWS = 112
CB_STEPS = N_PAD // 128 // CB_ROWS


def _combine_body(planes_ref, focal_ref, out_ref, acc):
  j = pl.program_id(0)

  @pl.when(j == 0)
  def _():
    acc[0] = 0.0
    acc[1] = 0.0
    acc[2] = 0.0

  in_s = planes_ref[0]
  out_s = planes_ref[NS]
  for k in range(1, NS):
    in_s = in_s + planes_ref[k]
    out_s = out_s + planes_ref[NS + k]

  n = (lax.broadcasted_iota(jnp.int32, (CB_ROWS, 128), 0) * 128
       + lax.broadcasted_iota(jnp.int32, (CB_ROWS, 128), 1)
       + j * (CB_ROWS * 128))
  customer = jnp.logical_and(n >= 1, n < N)
  zero = jnp.zeros_like(in_s)
  acc[0] += jnp.sum(jnp.where(customer, (in_s - 1.0) ** 2, zero)
                    + jnp.where(customer, (out_s - 1.0) ** 2, zero))
  diff = in_s - out_s
  acc[1] += jnp.sum(diff * diff)

  @pl.when(j == 0)
  def _():
    acc[2] = diff[0, 0] * diff[0, 0]

  @pl.when(j == CB_STEPS - 1)
  def _():
    out_ref[0, 0] = (COVERAGE_W * acc[0] / (2.0 * (N - 1))
                     + TOUR_W * acc[1] / N
                     + DEPOT_W * acc[2]
                     + SIM_W * focal_ref[0, 0] / E)


def _tc_combine(planes3, focal):
  return pl.pallas_call(
      _combine_body,
      grid=(CB_STEPS,),
      in_specs=[
          pl.BlockSpec((NW, CB_ROWS, 128), lambda j: (0, j, 0)),
          pl.BlockSpec(memory_space=pltpu.SMEM),
      ],
      out_specs=pl.BlockSpec(memory_space=pltpu.SMEM),
      out_shape=jax.ShapeDtypeStruct((1, 1), jnp.float32),
      scratch_shapes=[pltpu.SMEM((4,), jnp.float32)],
  )(planes3, focal)


def kernel(edge_predictions, edge_index, y_edges, num_nodes):
  preds2d = edge_predictions.reshape(EB, BLK)
  idx3d = edge_index.reshape(2, EB, BLK)
  y2d = y_edges.reshape(EB, BLK)

  focal, probs2d = _tc_focal(preds2d, y2d)
  planes = _sc_segment_sums(probs2d, idx3d)
  planes3 = planes.reshape(NW, N_PAD // 128, 128)
  total = _tc_combine(planes3, focal)
  return total.reshape(())


# speedup vs baseline: 45.0953x; 1.0739x over previous
import jax
import jax.numpy as jnp
from jax import lax
from jax.experimental import pallas as pl
from jax.experimental.pallas import tpu as pltpu
from jax.experimental.pallas import tpu_sc as plsc

N = 100000
E = 3200000
N_PAD = 100352
NC = 2
NS = 16
NW = NC * NS

BLK = 128
SUP = 40
EB = E // BLK
NCH = EB // SUP

COVERAGE_W = 5.0
TOUR_W = 3.0
DEPOT_W = 2.0
SIM_W = 0.3
FOCAL_ALPHA = 0.25
FOCAL_GAMMA = 2.0


TC_ROWS = 1000
TC_STEPS = EB // TC_ROWS


def _focal_body(preds_ref, y_ref, out_ref, probs_ref, acc):
  i = pl.program_id(0)

  @pl.when(i == 0)
  def _():
    acc[0] = 0.0

  x = preds_ref[...]
  t = y_ref[...]
  bce = jnp.maximum(x, 0.0) - x * t + jnp.log1p(jnp.exp(-jnp.abs(x)))
  probs = jax.nn.sigmoid(x)
  probs_ref[...] = probs
  p_t = probs * t + (1.0 - probs) * (1.0 - t)
  alpha_t = FOCAL_ALPHA * t + (1.0 - FOCAL_ALPHA) * (1.0 - t)
  w = 1.0 - p_t
  acc[0] += jnp.sum(alpha_t * (w * w) * bce)

  @pl.when(i == TC_STEPS - 1)
  def _():
    out_ref[0, 0] = acc[0]


def _tc_focal(preds2d, y2d):
  return pl.pallas_call(
      _focal_body,
      grid=(TC_STEPS,),
      in_specs=[
          pl.BlockSpec((TC_ROWS, 128), lambda i: (i, 0)),
          pl.BlockSpec((TC_ROWS, 128), lambda i: (i, 0)),
      ],
      out_specs=[
          pl.BlockSpec(memory_space=pltpu.SMEM),
          pl.BlockSpec((TC_ROWS, 128), lambda i: (i, 0)),
      ],
      out_shape=[
          jax.ShapeDtypeStruct((1, 1), jnp.float32),
          jax.ShapeDtypeStruct((EB, BLK), jnp.float32),
      ],
      scratch_shapes=[pltpu.SMEM((1,), jnp.float32)],
  )(preds2d, y2d)


def _sc_body(probs_hbm, idx_hbm, out_hbm,
             acc, pbuf0, pbuf1, ibuf0, ibuf1, sem0, sem1):
  c = lax.axis_index("c")
  s = lax.axis_index("s")
  wid = c * NS + s
  sel = 1 - c

  def _zero(i, _):
    acc[pl.ds(i * 16, 16)] = jnp.zeros((16,), jnp.float32)
    return 0
  lax.fori_loop(0, N_PAD // 16, _zero, 0, unroll=16)

  g0 = (s * NCH) // NS
  g1 = ((s + 1) * NCH) // NS

  def _load(g, pbuf, ibuf, sem):
    row = g * SUP
    pltpu.async_copy(probs_hbm.at[pl.ds(row, SUP), :], pbuf, sem)
    pltpu.async_copy(idx_hbm.at[sel, pl.ds(row * BLK, SUP * BLK)], ibuf, sem)

  def _wait(pbuf, ibuf, sem):
    pltpu.make_async_copy(probs_hbm.at[pl.ds(0, SUP), :], pbuf, sem).wait()
    pltpu.make_async_copy(idx_hbm.at[0, pl.ds(0, SUP * BLK)], ibuf, sem).wait()

  def _compute(pbuf, ibuf):
    def _row(r, _):
      for j in range(BLK // 16):
        p = pbuf[r, pl.ds(j * 16, 16)]
        ids = ibuf[pl.ds(r * BLK + j * 16, 16)]
        plsc.addupdate_scatter(acc, [ids], p)
      return 0
    lax.fori_loop(0, SUP, _row, 0, unroll=2)

  _load(g0, pbuf0, ibuf0, sem0)

  def _pair(k, _):
    a = g0 + 2 * k
    b = a + 1

    @pl.when(b < g1)
    def _():
      _load(b, pbuf1, ibuf1, sem1)

    _wait(pbuf0, ibuf0, sem0)
    _compute(pbuf0, ibuf0)

    @pl.when(a + 2 < g1)
    def _():
      _load(a + 2, pbuf0, ibuf0, sem0)

    @pl.when(b < g1)
    def _():
      _wait(pbuf1, ibuf1, sem1)
      _compute(pbuf1, ibuf1)
    return 0

  lax.fori_loop(0, (g1 - g0 + 1) // 2, _pair, 0)

  pltpu.sync_copy(acc, out_hbm.at[wid, 0])


def _sc_segment_sums(probs2d, idx3d):
  mesh = plsc.VectorSubcoreMesh(core_axis_name="c", subcore_axis_name="s")
  f = pl.kernel(
      _sc_body,
      out_type=jax.ShapeDtypeStruct((NW, 1, N_PAD), jnp.float32),
      mesh=mesh,
      compiler_params=pltpu.CompilerParams(
          needs_layout_passes=False, use_tc_tiling_on_sc=True),
      scratch_types=[
          pltpu.VMEM((N_PAD,), jnp.float32),
          pltpu.VMEM((SUP, BLK), jnp.float32),
          pltpu.VMEM((SUP, BLK), jnp.float32),
          pltpu.VMEM((SUP * BLK,), jnp.int32),
          pltpu.VMEM((SUP * BLK,), jnp.int32),
          pltpu.SemaphoreType.DMA,
          pltpu.SemaphoreType.DMA,
      ],
  )
  return f(probs2d, idx3d)


CB_ROWS = 112
CB_STEPS = N_PAD // 128 // CB_ROWS


def _combine_body(planes_ref, focal_ref, out_ref, acc):
  j = pl.program_id(0)

  @pl.when(j == 0)
  def _():
    acc[0] = 0.0
    acc[1] = 0.0
    acc[2] = 0.0

  in_s = planes_ref[0]
  out_s = planes_ref[NS]
  for k in range(1, NS):
    in_s = in_s + planes_ref[k]
    out_s = out_s + planes_ref[NS + k]

  n = (lax.broadcasted_iota(jnp.int32, (CB_ROWS, 128), 0) * 128
       + lax.broadcasted_iota(jnp.int32, (CB_ROWS, 128), 1)
       + j * (CB_ROWS * 128))
  customer = jnp.logical_and(n >= 1, n < N)
  zero = jnp.zeros_like(in_s)
  acc[0] += jnp.sum(jnp.where(customer, (in_s - 1.0) ** 2, zero)
                    + jnp.where(customer, (out_s - 1.0) ** 2, zero))
  diff = in_s - out_s
  acc[1] += jnp.sum(diff * diff)

  @pl.when(j == 0)
  def _():
    acc[2] = diff[0, 0] * diff[0, 0]

  @pl.when(j == CB_STEPS - 1)
  def _():
    out_ref[0, 0] = (COVERAGE_W * acc[0] / (2.0 * (N - 1))
                     + TOUR_W * acc[1] / N
                     + DEPOT_W * acc[2]
                     + SIM_W * focal_ref[0, 0] / E)


def _tc_combine(planes3, focal):
  return pl.pallas_call(
      _combine_body,
      grid=(CB_STEPS,),
      in_specs=[
          pl.BlockSpec((NW, CB_ROWS, 128), lambda j: (0, j, 0)),
          pl.BlockSpec(memory_space=pltpu.SMEM),
      ],
      out_specs=pl.BlockSpec(memory_space=pltpu.SMEM),
      out_shape=jax.ShapeDtypeStruct((1, 1), jnp.float32),
      scratch_shapes=[pltpu.SMEM((4,), jnp.float32)],
  )(planes3, focal)


def kernel(edge_predictions, edge_index, y_edges, num_nodes):
  preds2d = edge_predictions.reshape(EB, BLK)
  y2d = y_edges.reshape(EB, BLK)

  focal, probs2d = _tc_focal(preds2d, y2d)
  planes = _sc_segment_sums(probs2d, edge_index)
  planes3 = planes.reshape(NW, N_PAD // 128, 128)
  total = _tc_combine(planes3, focal)
  return total.reshape(())
